# baseline (speedup 1.0000x reference)
import functools

import jax
import jax.numpy as jnp
from jax import lax
from jax.experimental import pallas as pl
from jax.experimental.pallas import tpu as pltpu
from jax.experimental.pallas import tpu_sc as plsc

_VERTEX_IDS = (9120, 9929, 9448, 616, 6,
               5770, 5780, 8846, 8463, 8474, 8635,
               5361, 4933, 5058, 5169, 5286,
               8079, 7669, 7794, 7905, 8022)

_B = 1024
_J = 55
_E = len(_VERTEX_IDS)
_NW = 32
_BPW = _B // _NW


def _sc_body(verts_hbm, joints_hbm, out_hbm, jbuf, ebuf, sem):
    wid = lax.axis_index("s") * 2 + lax.axis_index("c")
    base = wid * _BPW

    copies = [pltpu.make_async_copy(
        joints_hbm.at[pl.ds(base, _BPW)], jbuf, sem)]
    copies += [
        pltpu.make_async_copy(
            verts_hbm.at[pl.ds(base, _BPW), pl.ds(idx, 1)],
            ebuf.at[:, pl.ds(j, 1)], sem)
        for j, idx in enumerate(_VERTEX_IDS)
    ]
    for c in copies:
        c.start()
    copies[0].wait()
    pltpu.sync_copy(jbuf, out_hbm.at[pl.ds(base, _BPW), pl.ds(0, _J)])
    for c in copies[1:]:
        c.wait()
    pltpu.sync_copy(ebuf, out_hbm.at[pl.ds(base, _BPW), pl.ds(_J, _E)])


def kernel(vertices, joints):
    mesh = plsc.VectorSubcoreMesh(core_axis_name="c", subcore_axis_name="s")
    run = functools.partial(
        pl.kernel,
        out_type=jax.ShapeDtypeStruct((_B, _J + _E, 3), jnp.float32),
        mesh=mesh,
        compiler_params=pltpu.CompilerParams(use_tc_tiling_on_sc=False),
        scratch_types=[
            pltpu.VMEM((_BPW, _J, 3), jnp.float32),
            pltpu.VMEM((_BPW, _E, 3), jnp.float32),
            pltpu.SemaphoreType.DMA,
        ],
    )(_sc_body)
    return run(vertices, joints)


# speedup vs baseline: 2013.4648x; 2013.4648x over previous
import functools

import jax
import jax.numpy as jnp
from jax import lax
from jax.experimental import pallas as pl
from jax.experimental.pallas import tpu as pltpu
from jax.experimental.pallas import tpu_sc as plsc

_VERTEX_IDS = (9120, 9929, 9448, 616, 6,
               5770, 5780, 8846, 8463, 8474, 8635,
               5361, 4933, 5058, 5169, 5286,
               8079, 7669, 7794, 7905, 8022)

_B = 1024
_V = 10475
_J = 55
_E = len(_VERTEX_IDS)
_NC = 8
_LANES = 16


def _sc_body(verts_hbm, joints_hbm, out_hbm, obuf, vstage, sem):
    wid = lax.axis_index("s") * 2 + lax.axis_index("c")

    @pl.when(wid < 3 * _NC)
    def _():
        c = wid // _NC
        col = (wid % _NC) * 128

        copies = [pltpu.make_async_copy(
            joints_hbm.at[c, :, pl.ds(col, 128)], obuf.at[pl.ds(0, _J)], sem)]
        copies += [
            pltpu.make_async_copy(
                verts_hbm.at[c, pl.ds(8 * (idx // 8), 8), pl.ds(col, 128)],
                vstage.at[j], sem)
            for j, idx in enumerate(_VERTEX_IDS)
        ]
        for cp in copies:
            cp.start()
        for cp in copies:
            cp.wait()

        for j, idx in enumerate(_VERTEX_IDS):
            r = idx % 8
            for p in range(128 // _LANES):
                obuf[_J + j, pl.ds(p * _LANES, _LANES)] = (
                    vstage[j, r, pl.ds(p * _LANES, _LANES)])

        pltpu.sync_copy(obuf, out_hbm.at[c, :, pl.ds(col, 128)])


def kernel(vertices, joints):
    vt = vertices.transpose(2, 1, 0)
    jt = joints.transpose(2, 1, 0)

    mesh = plsc.VectorSubcoreMesh(core_axis_name="c", subcore_axis_name="s")
    run = functools.partial(
        pl.kernel,
        out_type=jax.ShapeDtypeStruct((3, _J + _E, _B), jnp.float32),
        mesh=mesh,
        scratch_types=[
            pltpu.VMEM((_J + _E, 128), jnp.float32),
            pltpu.VMEM((_E, 8, 128), jnp.float32),
            pltpu.SemaphoreType.DMA,
        ],
    )(_sc_body)
    return run(vt, jt).transpose(2, 1, 0)


# speedup vs baseline: 14851.1952x; 7.3759x over previous
import jax
import jax.numpy as jnp
from jax.experimental import pallas as pl

_VERTEX_IDS = (9120, 9929, 9448, 616, 6,
               5770, 5780, 8846, 8463, 8474, 8635,
               5361, 4933, 5058, 5169, 5286,
               8079, 7669, 7794, 7905, 8022)

_B = 1024
_V = 10475
_J = 55
_E = len(_VERTEX_IDS)


def _body(jt_ref, *refs):
    stripe_refs = refs[:_E]
    out_ref = refs[_E]
    out_ref[:, : _J, :] = jt_ref[...]
    for j, idx in enumerate(_VERTEX_IDS):
        out_ref[:, _J + j, :] = stripe_refs[j][:, idx % 8, :]


def kernel(vertices, joints):
    vt = vertices.transpose(2, 1, 0)
    jt = joints.transpose(2, 1, 0)

    def stripe_spec(idx):
        blk = idx // 8
        return pl.BlockSpec((3, 8, _B), lambda i, blk=blk: (0, blk, 0))

    out_t = pl.pallas_call(
        _body,
        grid=(1,),
        out_shape=jax.ShapeDtypeStruct((3, _J + _E, _B), jnp.float32),
        in_specs=[pl.BlockSpec((3, _J, _B), lambda i: (0, 0, 0))]
        + [stripe_spec(idx) for idx in _VERTEX_IDS],
        out_specs=pl.BlockSpec((3, _J + _E, _B), lambda i: (0, 0, 0)),
    )(jt, *([vt] * _E))
    return out_t.transpose(2, 1, 0)
